# baseline (speedup 1.0000x reference)
import functools

import jax
import jax.numpy as jnp
from jax import lax
from jax.experimental import pallas as pl
from jax.experimental.pallas import tpu as pltpu
from jax.experimental.pallas import tpu_sc as plsc

N = 10000
E = 320000
F_IN = 128
H1 = 128
H2 = 64
H3 = 32

NC = 2
NS = 16
NW = NC * NS
EPT = E // NW
CH = 128
NCHP = 80
EPT_PAD = NCHP * CH
N_PAD = 10240
RPT = N_PAD // NS
ZR = 128

_mesh = lambda: plsc.VectorSubcoreMesh(core_axis_name="c", subcore_axis_name="s")


def _zero_vmem(ref, rows, width):
    z16 = jnp.zeros((16,), jnp.float32)

    def body(i, _):
        for j in range(width // 16):
            ref[i, pl.ds(j * 16, 16)] = z16
        return 0

    lax.fori_loop(0, rows, body, 0)


@functools.partial(
    pl.kernel,
    out_type=jax.ShapeDtypeStruct((NC * N_PAD, 16), jnp.float32),
    mesh=_mesh(),
    scratch_types=[
        pltpu.VMEM((NCHP, CH), jnp.int32),
        pltpu.VMEM((CH, 16), jnp.float32),
        pltpu.VMEM((RPT, 16), jnp.float32),
        pltpu.VMEM_SHARED((N_PAD, 16), jnp.float32),
        pltpu.SemaphoreType.DMA,
    ],
    compiler_params=pltpu.CompilerParams(use_tc_tiling_on_sc=False),
)
def _deg_kernel(dst2_hbm, out_hbm, di_v, ones_v, zst_v, acc_sh, sem):
    c = lax.axis_index("c")
    s = lax.axis_index("s")
    wid = s * NC + c

    one16 = jnp.ones((16,), jnp.float32)

    def initones(i, _):
        ones_v[i, :] = one16
        return 0

    lax.fori_loop(0, CH, initones, 0)
    _zero_vmem(zst_v, RPT, 16)
    pltpu.sync_copy(dst2_hbm.at[pl.ds(wid * NCHP, NCHP)], di_v)
    pltpu.sync_copy(zst_v, acc_sh.at[pl.ds(s * RPT, RPT)])
    plsc.subcore_barrier()

    GK = 8

    def body(g, _):
        for b in range(GK):
            pltpu.async_copy(ones_v, acc_sh.at[di_v.at[g * GK + b]], sem, add=True)
        for b in range(GK):
            pltpu.make_async_copy(ones_v, acc_sh.at[di_v.at[0]], sem).wait()
        return 0

    lax.fori_loop(0, NCHP // GK, body, 0)
    plsc.subcore_barrier()
    pltpu.sync_copy(acc_sh.at[pl.ds(s * RPT, RPT)], zst_v)
    pltpu.sync_copy(zst_v, out_hbm.at[pl.ds(c * N_PAD + s * RPT, RPT)])


NHALF = NCHP // 2


def _make_edge_kernel(F):
    @functools.partial(
        pl.kernel,
        out_type=jax.ShapeDtypeStruct((NC * N_PAD, F), jnp.float32),
        mesh=_mesh(),
        scratch_types=[
            pltpu.VMEM((NHALF, CH), jnp.int32),
            pltpu.VMEM((NHALF, CH), jnp.int32),
            pltpu.VMEM((CH, F), jnp.float32),
            pltpu.VMEM((CH, F), jnp.float32),
            pltpu.VMEM_SHARED((N_PAD, F), jnp.float32),
            pltpu.SemaphoreType.DMA,
            pltpu.SemaphoreType.DMA,
            pltpu.SemaphoreType.DMA,
            pltpu.SemaphoreType.DMA,
        ],
        compiler_params=pltpu.CompilerParams(use_tc_tiling_on_sc=False),
    )
    def ek(src2_hbm, dst2_hbm, t_hbm, out_hbm, si_v, di_v, rows0, rows1,
           acc_sh, semg0, semg1, sems0, sems1):
        c = lax.axis_index("c")
        s = lax.axis_index("s")
        wid = s * NC + c

        _zero_vmem(rows0, CH, F)
        for j in range(RPT // ZR):
            pltpu.sync_copy(rows0, acc_sh.at[pl.ds(s * RPT + j * ZR, ZR)])
        plsc.subcore_barrier()

        def start_g(buf, sem, i):
            pltpu.async_copy(t_hbm.at[si_v.at[i]], buf, sem)

        def wait_g(buf, sem):
            pltpu.make_async_copy(t_hbm.at[si_v.at[0]], buf, sem).wait()

        def start_s(buf, sem, i):
            pltpu.async_copy(buf, acc_sh.at[di_v.at[i]], sem, add=True)

        def wait_s(buf, sem):
            pltpu.make_async_copy(buf, acc_sh.at[di_v.at[0]], sem).wait()

        for h in range(NCHP // NHALF):
            base = wid * NCHP + h * NHALF
            pltpu.sync_copy(src2_hbm.at[pl.ds(base, NHALF)], si_v)
            pltpu.sync_copy(dst2_hbm.at[pl.ds(base, NHALF)], di_v)
            start_g(rows0, semg0, 0)
            start_g(rows1, semg1, 1)

            def body(k, _):
                i0 = 2 * k
                wait_g(rows0, semg0)
                start_s(rows0, sems0, i0)
                wait_g(rows1, semg1)
                start_s(rows1, sems1, i0 + 1)
                wait_s(rows0, sems0)
                start_g(rows0, semg0, i0 + 2)
                wait_s(rows1, sems1)
                start_g(rows1, semg1, i0 + 3)
                return 0

            lax.fori_loop(0, NHALF // 2 - 1, body, 0)
            wait_g(rows0, semg0)
            start_s(rows0, sems0, NHALF - 2)
            wait_g(rows1, semg1)
            start_s(rows1, sems1, NHALF - 1)
            wait_s(rows0, sems0)
            wait_s(rows1, sems1)

        plsc.subcore_barrier()
        for j in range(RPT // ZR):
            pltpu.sync_copy(acc_sh.at[pl.ds(s * RPT + j * ZR, ZR)], rows0)
            pltpu.sync_copy(
                rows0, out_hbm.at[pl.ds(c * N_PAD + s * RPT + j * ZR, ZR)]
            )

    return ek


_edge128 = _make_edge_kernel(H1)
_edge64 = _make_edge_kernel(H2)


def _dinvs(degp_ref):
    deg = (degp_ref[0, :N] + degp_ref[1, :N])[:, 0:1]
    dinv1 = lax.rsqrt(deg + 1.0)
    dinv2 = jnp.where(deg > 0, lax.rsqrt(jnp.maximum(deg, 1e-12)), 0.0)
    return dinv1, dinv2


def _tc1_body(degp_ref, x_ref, w1_ref, t1_ref):
    dinv1, _ = _dinvs(degp_ref)
    z = jnp.dot(x_ref[...], w1_ref[...], preferred_element_type=jnp.float32)
    t1_ref[...] = z * dinv1


def _tc2_body(degp_ref, s1p_ref, t1_ref, b1_ref, w2_ref, ws_ref, t2_ref, hs_ref):
    dinv1, dinv2 = _dinvs(degp_ref)
    h = jnp.maximum(
        dinv1 * (s1p_ref[0, :N] + s1p_ref[1, :N] + t1_ref[...]) + b1_ref[...], 0.0
    )
    t2_ref[...] = dinv2 * jnp.dot(h, w2_ref[...], preferred_element_type=jnp.float32)
    hs_ref[...] = jnp.dot(h, ws_ref[...], preferred_element_type=jnp.float32)


def _tc3_body(degp_ref, s2p_ref, hs_ref, b2_ref, wf1_ref, bf1_ref, wf2_ref, bf2_ref,
              out_ref):
    _, dinv2 = _dinvs(degp_ref)
    h2 = jnp.maximum(
        dinv2 * (s2p_ref[0, :N] + s2p_ref[1, :N]) + hs_ref[...] + b2_ref[...], 0.0
    )
    pooled = jnp.sum(h2, axis=0, keepdims=True)
    f = jnp.maximum(
        jnp.dot(pooled, wf1_ref[...], preferred_element_type=jnp.float32)
        + bf1_ref[...],
        0.0,
    )
    o = jnp.dot(f, wf2_ref[...], preferred_element_type=jnp.float32) + bf2_ref[...]
    out_ref[...] = 1.0 / (1.0 + jnp.exp(-o))


_tc1 = pl.pallas_call(_tc1_body, out_shape=jax.ShapeDtypeStruct((N, H1), jnp.float32))
_tc2 = pl.pallas_call(
    _tc2_body,
    out_shape=(
        jax.ShapeDtypeStruct((N, H2), jnp.float32),
        jax.ShapeDtypeStruct((N, H2), jnp.float32),
    ),
)
_tc3 = pl.pallas_call(_tc3_body, out_shape=jax.ShapeDtypeStruct((1, 1), jnp.float32))


def kernel(x, edge_index, W1, b1, W2, Ws, b2, Wf1, bf1, Wf2, bf2):
    src = edge_index[0]
    dst = edge_index[1]
    src2 = jnp.pad(src.reshape(NW, EPT), ((0, 0), (0, EPT_PAD - EPT))).reshape(
        NW * NCHP, CH
    )
    dst2 = jnp.pad(
        dst.reshape(NW, EPT), ((0, 0), (0, EPT_PAD - EPT)), constant_values=N
    ).reshape(NW * NCHP, CH)
    degp = _deg_kernel(dst2).reshape(NC, N_PAD, 16)
    t1 = _tc1(degp, x, W1)
    s1p = _edge128(src2, dst2, t1).reshape(NC, N_PAD, H1)
    t2, hs = _tc2(degp, s1p, t1, b1.reshape(1, H1), W2, Ws)
    s2p = _edge64(src2, dst2, t2).reshape(NC, N_PAD, H2)
    out = _tc3(
        degp, s2p, hs, b2.reshape(1, H2), Wf1, bf1.reshape(1, H3), Wf2,
        bf2.reshape(1, 1),
    )
    return out


# speedup vs baseline: 1.0705x; 1.0705x over previous
import functools

import jax
import jax.numpy as jnp
from jax import lax
from jax.experimental import pallas as pl
from jax.experimental.pallas import tpu as pltpu
from jax.experimental.pallas import tpu_sc as plsc

N = 10000
E = 320000
F_IN = 128
H1 = 128
H2 = 64
H3 = 32

NC = 2
NS = 16
NW = NC * NS
EPT = E // NW
CH = 128
NCHP = 80
EPT_PAD = NCHP * CH
N_PAD = 10240
RPT = N_PAD // NS
ZR = 128

_mesh = lambda: plsc.VectorSubcoreMesh(core_axis_name="c", subcore_axis_name="s")


def _zero_vmem(ref, rows, width):
    z16 = jnp.zeros((16,), jnp.float32)

    def body(i, _):
        for j in range(width // 16):
            ref[i, pl.ds(j * 16, 16)] = z16
        return 0

    lax.fori_loop(0, rows, body, 0)


@functools.partial(
    pl.kernel,
    out_type=jax.ShapeDtypeStruct((NC * N_PAD, 16), jnp.float32),
    mesh=_mesh(),
    scratch_types=[
        pltpu.VMEM((NCHP, CH), jnp.int32),
        pltpu.VMEM((CH, 16), jnp.float32),
        pltpu.VMEM((RPT, 16), jnp.float32),
        pltpu.VMEM_SHARED((N_PAD, 16), jnp.float32),
        pltpu.SemaphoreType.DMA,
    ],
    compiler_params=pltpu.CompilerParams(use_tc_tiling_on_sc=False),
)
def _deg_kernel(dst2_hbm, out_hbm, di_v, ones_v, zst_v, acc_sh, sem):
    c = lax.axis_index("c")
    s = lax.axis_index("s")
    wid = s * NC + c

    one16 = jnp.ones((16,), jnp.float32)

    def initones(i, _):
        ones_v[i, :] = one16
        return 0

    lax.fori_loop(0, CH, initones, 0)
    _zero_vmem(zst_v, RPT, 16)
    pltpu.sync_copy(dst2_hbm.at[pl.ds(wid * NCHP, NCHP)], di_v)
    pltpu.sync_copy(zst_v, acc_sh.at[pl.ds(s * RPT, RPT)])
    plsc.subcore_barrier()

    GK = 8

    def body(g, _):
        for b in range(GK):
            pltpu.async_copy(ones_v, acc_sh.at[di_v.at[g * GK + b]], sem, add=True)
        for b in range(GK):
            pltpu.make_async_copy(ones_v, acc_sh.at[di_v.at[0]], sem).wait()
        return 0

    lax.fori_loop(0, NCHP // GK, body, 0)
    plsc.subcore_barrier()
    pltpu.sync_copy(acc_sh.at[pl.ds(s * RPT, RPT)], zst_v)
    pltpu.sync_copy(zst_v, out_hbm.at[pl.ds(c * N_PAD + s * RPT, RPT)])


NHALF = NCHP // 2


def _make_edge_kernel(F):
    @functools.partial(
        pl.kernel,
        out_type=jax.ShapeDtypeStruct((NC * N_PAD, F), jnp.float32),
        mesh=_mesh(),
        scratch_types=[
            pltpu.VMEM((NHALF, CH), jnp.int32),
            pltpu.VMEM((NHALF, CH), jnp.int32),
            pltpu.VMEM((CH, F), jnp.float32),
            pltpu.VMEM((CH, F), jnp.float32),
            pltpu.VMEM_SHARED((N_PAD, F), jnp.float32),
            pltpu.SemaphoreType.DMA,
            pltpu.SemaphoreType.DMA,
            pltpu.SemaphoreType.DMA,
            pltpu.SemaphoreType.DMA,
        ],
        compiler_params=pltpu.CompilerParams(use_tc_tiling_on_sc=False),
    )
    def ek(src2_hbm, dst2_hbm, t_hbm, out_hbm, si_v, di_v, rows0, rows1,
           acc_sh, semg0, semg1, sems0, sems1):
        c = lax.axis_index("c")
        s = lax.axis_index("s")
        wid = s * NC + c

        _zero_vmem(rows0, CH, F)
        for j in range(RPT // ZR):
            pltpu.sync_copy(rows0, acc_sh.at[pl.ds(s * RPT + j * ZR, ZR)])
        plsc.subcore_barrier()

        def start_g(buf, sem, i):
            pltpu.async_copy(t_hbm.at[si_v.at[i]], buf, sem)

        def wait_g(buf, sem):
            pltpu.make_async_copy(t_hbm.at[si_v.at[0]], buf, sem).wait()

        def start_s(buf, sem, i):
            pltpu.async_copy(buf, acc_sh.at[di_v.at[i]], sem, add=True)

        def wait_s(buf, sem):
            pltpu.make_async_copy(buf, acc_sh.at[di_v.at[0]], sem).wait()

        for h in range(NCHP // NHALF):
            base = wid * NCHP + h * NHALF
            pltpu.sync_copy(src2_hbm.at[pl.ds(base, NHALF)], si_v)
            pltpu.sync_copy(dst2_hbm.at[pl.ds(base, NHALF)], di_v)
            start_g(rows0, semg0, 0)
            start_g(rows1, semg1, 1)

            def body(k, _):
                i0 = 2 * k
                wait_g(rows0, semg0)
                start_s(rows0, sems0, i0)
                wait_s(rows0, sems0)
                start_g(rows0, semg0, i0 + 2)
                wait_g(rows1, semg1)
                start_s(rows1, sems1, i0 + 1)
                wait_s(rows1, sems1)
                start_g(rows1, semg1, i0 + 3)
                return 0

            lax.fori_loop(0, NHALF // 2 - 1, body, 0)
            wait_g(rows0, semg0)
            start_s(rows0, sems0, NHALF - 2)
            wait_s(rows0, sems0)
            wait_g(rows1, semg1)
            start_s(rows1, sems1, NHALF - 1)
            wait_s(rows1, sems1)

        plsc.subcore_barrier()
        for j in range(RPT // ZR):
            pltpu.sync_copy(acc_sh.at[pl.ds(s * RPT + j * ZR, ZR)], rows0)
            pltpu.sync_copy(
                rows0, out_hbm.at[pl.ds(c * N_PAD + s * RPT + j * ZR, ZR)]
            )

    return ek


_edge128 = _make_edge_kernel(H1)
_edge64 = _make_edge_kernel(H2)


def _dinvs(degp_ref):
    deg = (degp_ref[0, :N] + degp_ref[1, :N])[:, 0:1]
    dinv1 = lax.rsqrt(deg + 1.0)
    dinv2 = jnp.where(deg > 0, lax.rsqrt(jnp.maximum(deg, 1e-12)), 0.0)
    return dinv1, dinv2


def _tc1_body(degp_ref, x_ref, w1_ref, t1_ref):
    dinv1, _ = _dinvs(degp_ref)
    z = jnp.dot(x_ref[...], w1_ref[...], preferred_element_type=jnp.float32)
    t1_ref[...] = z * dinv1


def _tc2_body(degp_ref, s1p_ref, t1_ref, b1_ref, w2_ref, ws_ref, t2_ref, hs_ref):
    dinv1, dinv2 = _dinvs(degp_ref)
    h = jnp.maximum(
        dinv1 * (s1p_ref[0, :N] + s1p_ref[1, :N] + t1_ref[...]) + b1_ref[...], 0.0
    )
    t2_ref[...] = dinv2 * jnp.dot(h, w2_ref[...], preferred_element_type=jnp.float32)
    hs_ref[...] = jnp.dot(h, ws_ref[...], preferred_element_type=jnp.float32)


def _tc3_body(degp_ref, s2p_ref, hs_ref, b2_ref, wf1_ref, bf1_ref, wf2_ref, bf2_ref,
              out_ref):
    _, dinv2 = _dinvs(degp_ref)
    h2 = jnp.maximum(
        dinv2 * (s2p_ref[0, :N] + s2p_ref[1, :N]) + hs_ref[...] + b2_ref[...], 0.0
    )
    pooled = jnp.sum(h2, axis=0, keepdims=True)
    f = jnp.maximum(
        jnp.dot(pooled, wf1_ref[...], preferred_element_type=jnp.float32)
        + bf1_ref[...],
        0.0,
    )
    o = jnp.dot(f, wf2_ref[...], preferred_element_type=jnp.float32) + bf2_ref[...]
    out_ref[...] = 1.0 / (1.0 + jnp.exp(-o))


_tc1 = pl.pallas_call(_tc1_body, out_shape=jax.ShapeDtypeStruct((N, H1), jnp.float32))
_tc2 = pl.pallas_call(
    _tc2_body,
    out_shape=(
        jax.ShapeDtypeStruct((N, H2), jnp.float32),
        jax.ShapeDtypeStruct((N, H2), jnp.float32),
    ),
)
_tc3 = pl.pallas_call(_tc3_body, out_shape=jax.ShapeDtypeStruct((1, 1), jnp.float32))


def kernel(x, edge_index, W1, b1, W2, Ws, b2, Wf1, bf1, Wf2, bf2):
    src = edge_index[0]
    dst = edge_index[1]
    src2 = jnp.pad(src.reshape(NW, EPT), ((0, 0), (0, EPT_PAD - EPT))).reshape(
        NW * NCHP, CH
    )
    dst2 = jnp.pad(
        dst.reshape(NW, EPT), ((0, 0), (0, EPT_PAD - EPT)), constant_values=N
    ).reshape(NW * NCHP, CH)
    degp = _deg_kernel(dst2).reshape(NC, N_PAD, 16)
    t1 = _tc1(degp, x, W1)
    s1p = _edge128(src2, dst2, t1).reshape(NC, N_PAD, H1)
    t2, hs = _tc2(degp, s1p, t1, b1.reshape(1, H1), W2, Ws)
    s2p = _edge64(src2, dst2, t2).reshape(NC, N_PAD, H2)
    out = _tc3(
        degp, s2p, hs, b2.reshape(1, H2), Wf1, bf1.reshape(1, H3), Wf2,
        bf2.reshape(1, 1),
    )
    return out


# speedup vs baseline: 1.6120x; 1.5058x over previous
import functools

import jax
import jax.numpy as jnp
from jax import lax
from jax.experimental import pallas as pl
from jax.experimental.pallas import tpu as pltpu
from jax.experimental.pallas import tpu_sc as plsc

N = 10000
E = 320000
F_IN = 128
H1 = 128
H2 = 64
H3 = 32

NC = 2
NS = 16
NW = NC * NS
EPT = E // NW
CH = 128
NCHP = 80
EPT_PAD = NCHP * CH
N_PAD = 10240
RPT = N_PAD // NS
ZR = 128

_mesh = lambda: plsc.VectorSubcoreMesh(core_axis_name="c", subcore_axis_name="s")


def _zero_vmem(ref, rows, width, dtype=jnp.float32):
    lanes = 16 if dtype == jnp.float32 else 32
    zv = jnp.zeros((lanes,), dtype)

    def body(i, _):
        for j in range(width // lanes):
            ref[i, pl.ds(j * lanes, lanes)] = zv
        return 0

    lax.fori_loop(0, rows, body, 0)


@functools.partial(
    pl.kernel,
    out_type=jax.ShapeDtypeStruct((NC * N_PAD, 16), jnp.float32),
    mesh=_mesh(),
    scratch_types=[
        pltpu.VMEM((NCHP, CH), jnp.int32),
        pltpu.VMEM((CH, 16), jnp.float32),
        pltpu.VMEM((RPT, 16), jnp.float32),
        pltpu.VMEM_SHARED((N_PAD, 16), jnp.float32),
        pltpu.SemaphoreType.DMA,
    ],
    compiler_params=pltpu.CompilerParams(use_tc_tiling_on_sc=False),
)
def _deg_kernel(dst2_hbm, out_hbm, di_v, ones_v, zst_v, acc_sh, sem):
    c = lax.axis_index("c")
    s = lax.axis_index("s")
    wid = s * NC + c

    one16 = jnp.ones((16,), jnp.float32)

    def initones(i, _):
        ones_v[i, :] = one16
        return 0

    lax.fori_loop(0, CH, initones, 0)
    _zero_vmem(zst_v, RPT, 16)
    pltpu.sync_copy(dst2_hbm.at[pl.ds(wid * NCHP, NCHP)], di_v)
    pltpu.sync_copy(zst_v, acc_sh.at[pl.ds(s * RPT, RPT)])
    plsc.subcore_barrier()

    GK = 8

    def body(g, _):
        for b in range(GK):
            pltpu.async_copy(ones_v, acc_sh.at[di_v.at[g * GK + b]], sem, add=True)
        for b in range(GK):
            pltpu.make_async_copy(ones_v, acc_sh.at[di_v.at[0]], sem).wait()
        return 0

    lax.fori_loop(0, NCHP // GK, body, 0)
    plsc.subcore_barrier()
    pltpu.sync_copy(acc_sh.at[pl.ds(s * RPT, RPT)], zst_v)
    pltpu.sync_copy(zst_v, out_hbm.at[pl.ds(c * N_PAD + s * RPT, RPT)])


NHALF = NCHP // 2


def _make_edge_kernel(F, dtype=jnp.float32):
    @functools.partial(
        pl.kernel,
        out_type=jax.ShapeDtypeStruct((NC * N_PAD, F), dtype),
        mesh=_mesh(),
        scratch_types=[
            pltpu.VMEM((NHALF, CH), jnp.int32),
            pltpu.VMEM((NHALF, CH), jnp.int32),
            pltpu.VMEM((CH, F), dtype),
            pltpu.VMEM((CH, F), dtype),
            pltpu.VMEM_SHARED((N_PAD, F), dtype),
            pltpu.SemaphoreType.DMA,
            pltpu.SemaphoreType.DMA,
            pltpu.SemaphoreType.DMA,
            pltpu.SemaphoreType.DMA,
        ],
        compiler_params=pltpu.CompilerParams(use_tc_tiling_on_sc=False),
    )
    def ek(src2_hbm, dst2_hbm, t_hbm, out_hbm, si_v, di_v, rows0, rows1,
           acc_sh, semg0, semg1, sems0, sems1):
        c = lax.axis_index("c")
        s = lax.axis_index("s")
        wid = s * NC + c

        _zero_vmem(rows0, CH, F, dtype)
        for j in range(RPT // ZR):
            pltpu.sync_copy(rows0, acc_sh.at[pl.ds(s * RPT + j * ZR, ZR)])
        plsc.subcore_barrier()

        def start_g(buf, sem, i):
            pltpu.async_copy(t_hbm.at[si_v.at[i]], buf, sem)

        def wait_g(buf, sem):
            pltpu.make_async_copy(t_hbm.at[si_v.at[0]], buf, sem).wait()

        def start_s(buf, sem, i):
            pltpu.async_copy(buf, acc_sh.at[di_v.at[i]], sem, add=True)

        def wait_s(buf, sem):
            pltpu.make_async_copy(buf, acc_sh.at[di_v.at[0]], sem).wait()

        for h in range(NCHP // NHALF):
            base = wid * NCHP + h * NHALF
            pltpu.sync_copy(src2_hbm.at[pl.ds(base, NHALF)], si_v)
            pltpu.sync_copy(dst2_hbm.at[pl.ds(base, NHALF)], di_v)
            start_g(rows0, semg0, 0)
            start_g(rows1, semg1, 1)

            def body(k, _):
                i0 = 2 * k
                wait_g(rows0, semg0)
                start_s(rows0, sems0, i0)
                wait_s(rows0, sems0)
                start_g(rows0, semg0, i0 + 2)
                wait_g(rows1, semg1)
                start_s(rows1, sems1, i0 + 1)
                wait_s(rows1, sems1)
                start_g(rows1, semg1, i0 + 3)
                return 0

            lax.fori_loop(0, NHALF // 2 - 1, body, 0)
            wait_g(rows0, semg0)
            start_s(rows0, sems0, NHALF - 2)
            wait_s(rows0, sems0)
            wait_g(rows1, semg1)
            start_s(rows1, sems1, NHALF - 1)
            wait_s(rows1, sems1)

        plsc.subcore_barrier()
        for j in range(RPT // ZR):
            pltpu.sync_copy(acc_sh.at[pl.ds(s * RPT + j * ZR, ZR)], rows0)
            pltpu.sync_copy(
                rows0, out_hbm.at[pl.ds(c * N_PAD + s * RPT + j * ZR, ZR)]
            )

    return ek


_edge128 = _make_edge_kernel(H1, jnp.bfloat16)
_edge64 = _make_edge_kernel(H2, jnp.bfloat16)


def _dinvs(degp_ref):
    deg = (degp_ref[0, :N] + degp_ref[1, :N])[:, 0:1]
    dinv1 = lax.rsqrt(deg + 1.0)
    dinv2 = jnp.where(deg > 0, lax.rsqrt(jnp.maximum(deg, 1e-12)), 0.0)
    return dinv1, dinv2


def _tc1_body(degp_ref, x_ref, w1_ref, t1_ref, t1b_ref):
    dinv1, _ = _dinvs(degp_ref)
    z = jnp.dot(x_ref[...], w1_ref[...], preferred_element_type=jnp.float32)
    t1 = z * dinv1
    t1_ref[...] = t1
    t1b_ref[...] = t1.astype(jnp.bfloat16)


def _tc2_body(degp_ref, s1p_ref, t1_ref, b1_ref, w2_ref, ws_ref, t2b_ref, hs_ref):
    dinv1, dinv2 = _dinvs(degp_ref)
    s1 = s1p_ref[0, :N].astype(jnp.float32) + s1p_ref[1, :N].astype(jnp.float32)
    h = jnp.maximum(dinv1 * (s1 + t1_ref[...]) + b1_ref[...], 0.0)
    t2 = dinv2 * jnp.dot(h, w2_ref[...], preferred_element_type=jnp.float32)
    t2b_ref[...] = t2.astype(jnp.bfloat16)
    hs_ref[...] = jnp.dot(h, ws_ref[...], preferred_element_type=jnp.float32)


def _tc3_body(degp_ref, s2p_ref, hs_ref, b2_ref, wf1_ref, bf1_ref, wf2_ref, bf2_ref,
              out_ref):
    _, dinv2 = _dinvs(degp_ref)
    s2 = s2p_ref[0, :N].astype(jnp.float32) + s2p_ref[1, :N].astype(jnp.float32)
    h2 = jnp.maximum(dinv2 * s2 + hs_ref[...] + b2_ref[...], 0.0)
    pooled = jnp.sum(h2, axis=0, keepdims=True)
    f = jnp.maximum(
        jnp.dot(pooled, wf1_ref[...], preferred_element_type=jnp.float32)
        + bf1_ref[...],
        0.0,
    )
    o = jnp.dot(f, wf2_ref[...], preferred_element_type=jnp.float32) + bf2_ref[...]
    out_ref[...] = 1.0 / (1.0 + jnp.exp(-o))


_tc1 = pl.pallas_call(
    _tc1_body,
    out_shape=(
        jax.ShapeDtypeStruct((N, H1), jnp.float32),
        jax.ShapeDtypeStruct((N, H1), jnp.bfloat16),
    ),
)
_tc2 = pl.pallas_call(
    _tc2_body,
    out_shape=(
        jax.ShapeDtypeStruct((N, H2), jnp.bfloat16),
        jax.ShapeDtypeStruct((N, H2), jnp.float32),
    ),
)
_tc3 = pl.pallas_call(_tc3_body, out_shape=jax.ShapeDtypeStruct((1, 1), jnp.float32))


def kernel(x, edge_index, W1, b1, W2, Ws, b2, Wf1, bf1, Wf2, bf2):
    src = edge_index[0]
    dst = edge_index[1]
    src2 = jnp.pad(src.reshape(NW, EPT), ((0, 0), (0, EPT_PAD - EPT))).reshape(
        NW * NCHP, CH
    )
    dst2 = jnp.pad(
        dst.reshape(NW, EPT), ((0, 0), (0, EPT_PAD - EPT)), constant_values=N
    ).reshape(NW * NCHP, CH)
    degp = _deg_kernel(dst2).reshape(NC, N_PAD, 16)
    t1, t1b = _tc1(degp, x, W1)
    s1p = _edge128(src2, dst2, t1b).reshape(NC, N_PAD, H1)
    t2b, hs = _tc2(degp, s1p, t1, b1.reshape(1, H1), W2, Ws)
    s2p = _edge64(src2, dst2, t2b).reshape(NC, N_PAD, H2)
    out = _tc3(
        degp, s2p, hs, b2.reshape(1, H2), Wf1, bf1.reshape(1, H3), Wf2,
        bf2.reshape(1, 1),
    )
    return out


# speedup vs baseline: 2.8380x; 1.7606x over previous
import functools

import jax
import jax.numpy as jnp
from jax import lax
from jax.experimental import pallas as pl
from jax.experimental.pallas import tpu as pltpu
from jax.experimental.pallas import tpu_sc as plsc

N = 10000
E = 320000
F_IN = 128
H1 = 128
H2 = 64
H3 = 32

NC = 2
NS = 16
NW = NC * NS
EPT = E // NW
CH = 128
NCHP = 80
EPT_PAD = NCHP * CH
N_PAD = 10240
RPT = N_PAD // NS
ZR = 128

_mesh = lambda: plsc.VectorSubcoreMesh(core_axis_name="c", subcore_axis_name="s")


def _zero_vmem(ref, rows, width, dtype=jnp.float32):
    lanes = 16 if dtype == jnp.float32 else 32
    zv = jnp.zeros((lanes,), dtype)

    def body(i, _):
        for j in range(width // lanes):
            ref[i, pl.ds(j * lanes, lanes)] = zv
        return 0

    lax.fori_loop(0, rows, body, 0)


@functools.partial(
    pl.kernel,
    out_type=jax.ShapeDtypeStruct((NC * N_PAD, 16), jnp.float32),
    mesh=_mesh(),
    scratch_types=[
        pltpu.VMEM((NCHP, CH), jnp.int32),
        pltpu.VMEM((CH, 16), jnp.float32),
        pltpu.VMEM((RPT, 16), jnp.float32),
        pltpu.VMEM_SHARED((N_PAD, 16), jnp.float32),
        pltpu.SemaphoreType.DMA,
    ],
    compiler_params=pltpu.CompilerParams(use_tc_tiling_on_sc=False),
)
def _deg_kernel(dst2_hbm, out_hbm, di_v, ones_v, zst_v, acc_sh, sem):
    c = lax.axis_index("c")
    s = lax.axis_index("s")
    wid = s * NC + c

    one16 = jnp.ones((16,), jnp.float32)

    def initones(i, _):
        ones_v[i, :] = one16
        return 0

    lax.fori_loop(0, CH, initones, 0)
    _zero_vmem(zst_v, RPT, 16)
    pltpu.sync_copy(dst2_hbm.at[pl.ds(wid * NCHP, NCHP)], di_v)
    pltpu.sync_copy(zst_v, acc_sh.at[pl.ds(s * RPT, RPT)])
    plsc.subcore_barrier()

    GK = 8

    def body(g, _):
        for b in range(GK):
            pltpu.async_copy(ones_v, acc_sh.at[di_v.at[g * GK + b]], sem, add=True)
        for b in range(GK):
            pltpu.make_async_copy(ones_v, acc_sh.at[di_v.at[0]], sem).wait()
        return 0

    lax.fori_loop(0, NCHP // GK, body, 0)
    plsc.subcore_barrier()
    pltpu.sync_copy(acc_sh.at[pl.ds(s * RPT, RPT)], zst_v)
    pltpu.sync_copy(zst_v, out_hbm.at[pl.ds(c * N_PAD + s * RPT, RPT)])


NHALF = NCHP // 2


def _make_edge_kernel(F, dtype=jnp.float32):
    @functools.partial(
        pl.kernel,
        out_type=jax.ShapeDtypeStruct((NC * N_PAD, F), dtype),
        mesh=_mesh(),
        scratch_types=[
            pltpu.VMEM((NHALF, CH), jnp.int32),
            pltpu.VMEM((NHALF, CH), jnp.int32),
            pltpu.VMEM((CH, F), dtype),
            pltpu.VMEM((CH, F), dtype),
            pltpu.VMEM_SHARED((N_PAD, F), dtype),
            pltpu.VMEM_SHARED((N_PAD, F), dtype),
            pltpu.SemaphoreType.DMA,
            pltpu.SemaphoreType.DMA,
            pltpu.SemaphoreType.DMA,
            pltpu.SemaphoreType.DMA,
        ],
        compiler_params=pltpu.CompilerParams(use_tc_tiling_on_sc=False),
    )
    def ek(src2_hbm, dst2_hbm, t_hbm, out_hbm, si_v, di_v, rows0, rows1,
           acc_sh, t_sh, semg0, semg1, sems0, sems1):
        c = lax.axis_index("c")
        s = lax.axis_index("s")
        wid = s * NC + c

        pltpu.sync_copy(t_hbm.at[pl.ds(s * RPT, RPT)], t_sh.at[pl.ds(s * RPT, RPT)])
        _zero_vmem(rows0, CH, F, dtype)
        for j in range(RPT // ZR):
            pltpu.sync_copy(rows0, acc_sh.at[pl.ds(s * RPT + j * ZR, ZR)])
        plsc.subcore_barrier()

        def start_g(buf, sem, i):
            pltpu.async_copy(t_sh.at[si_v.at[i]], buf, sem)

        def wait_g(buf, sem):
            pltpu.make_async_copy(t_sh.at[si_v.at[0]], buf, sem).wait()

        def start_s(buf, sem, i):
            pltpu.async_copy(buf, acc_sh.at[di_v.at[i]], sem, add=True)

        def wait_s(buf, sem):
            pltpu.make_async_copy(buf, acc_sh.at[di_v.at[0]], sem).wait()

        for h in range(NCHP // NHALF):
            base = wid * NCHP + h * NHALF
            pltpu.sync_copy(src2_hbm.at[pl.ds(base, NHALF)], si_v)
            pltpu.sync_copy(dst2_hbm.at[pl.ds(base, NHALF)], di_v)
            start_g(rows0, semg0, 0)
            start_g(rows1, semg1, 1)

            def body(k, _):
                i0 = 2 * k
                wait_g(rows0, semg0)
                start_s(rows0, sems0, i0)
                wait_s(rows0, sems0)
                start_g(rows0, semg0, i0 + 2)
                wait_g(rows1, semg1)
                start_s(rows1, sems1, i0 + 1)
                wait_s(rows1, sems1)
                start_g(rows1, semg1, i0 + 3)
                return 0

            lax.fori_loop(0, NHALF // 2 - 1, body, 0)
            wait_g(rows0, semg0)
            start_s(rows0, sems0, NHALF - 2)
            wait_s(rows0, sems0)
            wait_g(rows1, semg1)
            start_s(rows1, sems1, NHALF - 1)
            wait_s(rows1, sems1)

        plsc.subcore_barrier()
        for j in range(RPT // ZR):
            pltpu.sync_copy(acc_sh.at[pl.ds(s * RPT + j * ZR, ZR)], rows0)
            pltpu.sync_copy(
                rows0, out_hbm.at[pl.ds(c * N_PAD + s * RPT + j * ZR, ZR)]
            )

    return ek


_edge128 = _make_edge_kernel(H1, jnp.bfloat16)
_edge64 = _make_edge_kernel(H2, jnp.bfloat16)


def _dinvs(degp_ref):
    deg = (degp_ref[0, :N] + degp_ref[1, :N])[:, 0:1]
    dinv1 = lax.rsqrt(deg + 1.0)
    dinv2 = jnp.where(deg > 0, lax.rsqrt(jnp.maximum(deg, 1e-12)), 0.0)
    return dinv1, dinv2


def _tc1_body(degp_ref, x_ref, w1_ref, t1_ref, t1b_ref):
    dinv1, _ = _dinvs(degp_ref)
    z = jnp.dot(x_ref[...], w1_ref[...], preferred_element_type=jnp.float32)
    t1 = z * dinv1
    t1_ref[...] = t1
    t1b_ref[...] = t1.astype(jnp.bfloat16)


def _tc2_body(degp_ref, s1p_ref, t1_ref, b1_ref, w2_ref, ws_ref, t2b_ref, hs_ref):
    dinv1, dinv2 = _dinvs(degp_ref)
    s1 = s1p_ref[0, :N].astype(jnp.float32) + s1p_ref[1, :N].astype(jnp.float32)
    h = jnp.maximum(dinv1 * (s1 + t1_ref[...]) + b1_ref[...], 0.0)
    t2 = dinv2 * jnp.dot(h, w2_ref[...], preferred_element_type=jnp.float32)
    t2b_ref[...] = t2.astype(jnp.bfloat16)
    hs_ref[...] = jnp.dot(h, ws_ref[...], preferred_element_type=jnp.float32)


def _tc3_body(degp_ref, s2p_ref, hs_ref, b2_ref, wf1_ref, bf1_ref, wf2_ref, bf2_ref,
              out_ref):
    _, dinv2 = _dinvs(degp_ref)
    s2 = s2p_ref[0, :N].astype(jnp.float32) + s2p_ref[1, :N].astype(jnp.float32)
    h2 = jnp.maximum(dinv2 * s2 + hs_ref[...] + b2_ref[...], 0.0)
    pooled = jnp.sum(h2, axis=0, keepdims=True)
    f = jnp.maximum(
        jnp.dot(pooled, wf1_ref[...], preferred_element_type=jnp.float32)
        + bf1_ref[...],
        0.0,
    )
    o = jnp.dot(f, wf2_ref[...], preferred_element_type=jnp.float32) + bf2_ref[...]
    out_ref[...] = 1.0 / (1.0 + jnp.exp(-o))


_tc1 = pl.pallas_call(
    _tc1_body,
    out_shape=(
        jax.ShapeDtypeStruct((N, H1), jnp.float32),
        jax.ShapeDtypeStruct((N, H1), jnp.bfloat16),
    ),
)
_tc2 = pl.pallas_call(
    _tc2_body,
    out_shape=(
        jax.ShapeDtypeStruct((N, H2), jnp.bfloat16),
        jax.ShapeDtypeStruct((N, H2), jnp.float32),
    ),
)
_tc3 = pl.pallas_call(_tc3_body, out_shape=jax.ShapeDtypeStruct((1, 1), jnp.float32))


def kernel(x, edge_index, W1, b1, W2, Ws, b2, Wf1, bf1, Wf2, bf2):
    src = edge_index[0]
    dst = edge_index[1]
    src2 = jnp.pad(src.reshape(NW, EPT), ((0, 0), (0, EPT_PAD - EPT))).reshape(
        NW * NCHP, CH
    )
    dst2 = jnp.pad(
        dst.reshape(NW, EPT), ((0, 0), (0, EPT_PAD - EPT)), constant_values=N
    ).reshape(NW * NCHP, CH)
    degp = _deg_kernel(dst2).reshape(NC, N_PAD, 16)
    t1, t1b = _tc1(degp, x, W1)
    t1b = jnp.pad(t1b, ((0, N_PAD - N), (0, 0)))
    s1p = _edge128(src2, dst2, t1b).reshape(NC, N_PAD, H1)
    t2b, hs = _tc2(degp, s1p, t1, b1.reshape(1, H1), W2, Ws)
    t2b = jnp.pad(t2b, ((0, N_PAD - N), (0, 0)))
    s2p = _edge64(src2, dst2, t2b).reshape(NC, N_PAD, H2)
    out = _tc3(
        degp, s2p, hs, b2.reshape(1, H2), Wf1, bf1.reshape(1, H3), Wf2,
        bf2.reshape(1, 1),
    )
    return out
